# Initial kernel scaffold; baseline (speedup 1.0000x reference)
import jax
import jax.numpy as jnp
from jax.experimental import pallas as pl


def kernel(x, edge_index, W1, b1, W2, b2):
    raise NotImplementedError("write your pallas kernel here")


# speedup vs baseline: 18.4867x; 18.4867x over previous
import functools

import jax
import jax.numpy as jnp
from jax import lax
from jax.experimental import pallas as pl
from jax.experimental.pallas import tpu as pltpu
from jax.experimental.pallas import tpu_sc as plsc

N = 10000
E = 320000
D = 128

NC = 2
NS = 16
NW = NC * NS
NP = 10240
ROWS_PER_SUB = NP // NS
EDGES_PER_W = E // NW

EDGE_C = 200
EDGE_CHUNKS = EDGES_PER_W // EDGE_C

DEG_C = 200
DEG_CHUNKS = EDGES_PER_W // DEG_C

_MESH = plsc.VectorSubcoreMesh(core_axis_name="c", subcore_axis_name="s")

ROW_B = 1000
GRID = N // ROW_B


def _deg_body(dst_hbm, zeros_hbm, ones_hbm, out_hbm, idx_v, ones_v, acc_sh,
              sem):
    c = lax.axis_index("c")
    s = lax.axis_index("s")
    wid = c * NS + s
    pltpu.sync_copy(zeros_hbm, acc_sh.at[pl.ds(s * ROWS_PER_SUB, ROWS_PER_SUB)])
    pltpu.sync_copy(ones_hbm, ones_v)
    plsc.subcore_barrier()

    @pl.loop(0, DEG_CHUNKS)
    def _(i):
        base = wid * EDGES_PER_W + i * DEG_C
        pltpu.sync_copy(dst_hbm.at[pl.ds(base, DEG_C)], idx_v)
        pltpu.sync_copy(ones_v, acc_sh.at[idx_v], add=True)

    plsc.subcore_barrier()
    pltpu.sync_copy(acc_sh.at[pl.ds(s * ROWS_PER_SUB, ROWS_PER_SUB)],
                    out_hbm.at[c].at[pl.ds(s * ROWS_PER_SUB, ROWS_PER_SUB)])


def _degree_histogram(dst):
    zeros = jnp.zeros((ROWS_PER_SUB, 16), jnp.float32)
    ones = jnp.ones((DEG_C, 16), jnp.float32)
    return pl.kernel(
        _deg_body,
        out_type=jax.ShapeDtypeStruct((NC, NP, 16), jnp.float32),
        mesh=_MESH,
        scratch_types=[
            pltpu.VMEM((DEG_C,), jnp.int32),
            pltpu.VMEM((DEG_C, 16), jnp.float32),
            pltpu.VMEM_SHARED((NP, 16), jnp.float32),
            pltpu.SemaphoreType.DMA,
        ],
    )(dst, zeros, ones)


def _agg_body(g_hbm, src_hbm, dst_hbm, zeros_hbm, out_hbm, src_v, dst_v,
              rows_v, acc_sh, sem):
    c = lax.axis_index("c")
    s = lax.axis_index("s")
    wid = c * NS + s
    pltpu.sync_copy(zeros_hbm, acc_sh.at[pl.ds(s * ROWS_PER_SUB, ROWS_PER_SUB)])
    plsc.subcore_barrier()

    @pl.loop(0, EDGE_CHUNKS)
    def _(i):
        base = wid * EDGES_PER_W + i * EDGE_C
        pltpu.sync_copy(src_hbm.at[pl.ds(base, EDGE_C)], src_v)
        pltpu.sync_copy(dst_hbm.at[pl.ds(base, EDGE_C)], dst_v)
        pltpu.async_copy(g_hbm.at[src_v], rows_v, sem).wait()
        pltpu.sync_copy(rows_v, acc_sh.at[dst_v], add=True)

    plsc.subcore_barrier()
    pltpu.sync_copy(acc_sh.at[pl.ds(s * ROWS_PER_SUB, ROWS_PER_SUB)],
                    out_hbm.at[c].at[pl.ds(s * ROWS_PER_SUB, ROWS_PER_SUB)])


def _aggregate(g, src, dst):
    zeros = jnp.zeros((ROWS_PER_SUB, D), jnp.float32)
    return pl.kernel(
        _agg_body,
        out_type=jax.ShapeDtypeStruct((NC, NP, D), jnp.float32),
        mesh=_MESH,
        scratch_types=[
            pltpu.VMEM((EDGE_C,), jnp.int32),
            pltpu.VMEM((EDGE_C,), jnp.int32),
            pltpu.VMEM((EDGE_C, D), jnp.float32),
            pltpu.VMEM_SHARED((NP, D), jnp.float32),
            pltpu.SemaphoreType.DMA,
        ],
    )(g, src, dst, zeros)


def _pre_body(x_ref, dparts_ref, w1_ref, g1_ref, dinv_ref):
    deg = dparts_ref[0] + dparts_ref[1] + 1.0
    dinv = lax.rsqrt(deg)
    h = jnp.dot(x_ref[...], w1_ref[...], preferred_element_type=jnp.float32)
    g1_ref[...] = dinv[:, :1] * h
    dinv_ref[...] = dinv


def _pre(x, dparts, W1):
    return pl.pallas_call(
        _pre_body,
        grid=(GRID,),
        in_specs=[
            pl.BlockSpec((ROW_B, D), lambda i: (i, 0)),
            pl.BlockSpec((NC, ROW_B, 16), lambda i: (0, i, 0)),
            pl.BlockSpec((D, D), lambda i: (0, 0)),
        ],
        out_specs=[
            pl.BlockSpec((ROW_B, D), lambda i: (i, 0)),
            pl.BlockSpec((ROW_B, 16), lambda i: (i, 0)),
        ],
        out_shape=[
            jax.ShapeDtypeStruct((N, D), jnp.float32),
            jax.ShapeDtypeStruct((N, 16), jnp.float32),
        ],
    )(x, dparts, W1)


def _mid_body(parts_ref, g1_ref, dinv_ref, b_ref, w2_ref, g2_ref):
    dinv = dinv_ref[:, :1]
    z = dinv * (parts_ref[0] + parts_ref[1] + g1_ref[...]) + b_ref[...]
    h = jnp.maximum(z, 0.0)
    g2_ref[...] = dinv * jnp.dot(h, w2_ref[...],
                                 preferred_element_type=jnp.float32)


def _mid(parts, g1, dinv, b1, W2):
    return pl.pallas_call(
        _mid_body,
        grid=(GRID,),
        in_specs=[
            pl.BlockSpec((NC, ROW_B, D), lambda i: (0, i, 0)),
            pl.BlockSpec((ROW_B, D), lambda i: (i, 0)),
            pl.BlockSpec((ROW_B, 16), lambda i: (i, 0)),
            pl.BlockSpec((1, D), lambda i: (0, 0)),
            pl.BlockSpec((D, D), lambda i: (0, 0)),
        ],
        out_specs=pl.BlockSpec((ROW_B, D), lambda i: (i, 0)),
        out_shape=jax.ShapeDtypeStruct((N, D), jnp.float32),
    )(parts, g1, dinv, b1, W2)


def _fin_body(parts_ref, g2_ref, dinv_ref, b_ref, out_ref):
    dinv = dinv_ref[:, :1]
    z = dinv * (parts_ref[0] + parts_ref[1] + g2_ref[...]) + b_ref[...]
    out_ref[...] = jnp.maximum(z, 0.0)


def _fin(parts, g2, dinv, b2):
    return pl.pallas_call(
        _fin_body,
        grid=(GRID,),
        in_specs=[
            pl.BlockSpec((NC, ROW_B, D), lambda i: (0, i, 0)),
            pl.BlockSpec((ROW_B, D), lambda i: (i, 0)),
            pl.BlockSpec((ROW_B, 16), lambda i: (i, 0)),
            pl.BlockSpec((1, D), lambda i: (0, 0)),
        ],
        out_specs=pl.BlockSpec((ROW_B, D), lambda i: (i, 0)),
        out_shape=jax.ShapeDtypeStruct((N, D), jnp.float32),
    )(parts, g2, dinv, b2)


def kernel(x, edge_index, W1, b1, W2, b2):
    src = edge_index[0]
    dst = edge_index[1]
    dparts = _degree_histogram(dst)[:, :N]
    g1, dinv = _pre(x, dparts, W1)
    aparts = _aggregate(g1, src, dst)[:, :N]
    g2 = _mid(aparts, g1, dinv, b1.reshape(1, D), W2)
    qparts = _aggregate(g2, src, dst)[:, :N]
    return _fin(qparts, g2, dinv, b2.reshape(1, D))
